# Initial kernel scaffold; baseline (speedup 1.0000x reference)
import jax
import jax.numpy as jnp
from jax.experimental import pallas as pl


def kernel(loc_preds, loc_targets, cls_preds, cls_targets):
    raise NotImplementedError("write your pallas kernel here")


# speedup vs baseline: 2.1112x; 2.1112x over previous
import jax
import jax.numpy as jnp
from jax import lax
from jax.experimental import pallas as pl
from jax.experimental.pallas import tpu as pltpu

N, A, C = 64, 8732, 81


def _body(lp_ref, lt_ref, cp_ref, tt_ref, out_ref,
          key_s, mval_s, k_s, locrow_s, poscls_s):
    n = pl.program_id(0)
    t = tt_ref[0]
    pos = t > 0
    posf = pos.astype(jnp.float32)

    cp = cp_ref[0]
    cidx = lax.broadcasted_iota(jnp.int32, (C, 1), 0)
    g = jnp.sum(jnp.where(cidx == t, cp, 0.0), axis=0, keepdims=True)

    d = lp_ref[...] - lt_ref[...]
    ad = jnp.abs(d)
    sl1 = jnp.where(ad < 1.0, 0.5 * d * d, ad - 0.5)
    loc_row = jnp.sum(sl1 * posf[:, :, None])

    pcnt = jnp.sum(pos.astype(jnp.int32))
    poscls_row = jnp.sum(jnp.where(pos, -g, 0.0))

    masked = jnp.where(pos, 0.0, g)
    u = lax.bitcast_convert_type(masked, jnp.uint32)
    neg_sign = u >= jnp.uint32(0x80000000)
    key = jnp.where(neg_sign, ~u, u ^ jnp.uint32(0x80000000))

    key_s[pl.ds(n, 1), :] = key
    mval_s[pl.ds(n, 1), :] = masked
    k_s[pl.ds(n, 1), :] = (3 * pcnt)[None, None]
    locrow_s[pl.ds(n, 1), :] = loc_row[None, None]
    poscls_s[pl.ds(n, 1), :] = poscls_row[None, None]

    @pl.when(n == N - 1)
    def _():
        key = key_s[...]
        mval = mval_s[...]
        k_raw = k_s[...]
        k_eff = jnp.minimum(k_raw, A)
        kr0 = jnp.maximum(k_eff, 1)

        def bit_step(i, carry):
            prefix, kr = carry
            b = (31 - i).astype(jnp.uint32)
            cond = (key >> b) == (prefix >> b)
            c = jnp.sum(cond.astype(jnp.int32), axis=1, keepdims=True)
            take1 = kr > c
            prefix = jnp.where(take1, prefix | (jnp.uint32(1) << b), prefix)
            kr = jnp.where(take1, kr - c, kr)
            return prefix, kr

        prefix, _ = lax.fori_loop(
            0, 32, bit_step, (jnp.zeros((N, 1), jnp.uint32), kr0))

        T = prefix
        less = key < T
        count_less = jnp.sum(less.astype(jnp.int32), axis=1, keepdims=True)
        sum_less = jnp.sum(jnp.where(less, mval, 0.0), axis=1, keepdims=True)
        neg_t = T < jnp.uint32(0x80000000)
        uT = jnp.where(neg_t, ~T, T ^ jnp.uint32(0x80000000))
        tval = lax.bitcast_convert_type(uT, jnp.float32)
        sel = sum_less + (k_eff - count_less).astype(jnp.float32) * tval
        sel = jnp.where(k_eff <= 0, 0.0, sel)

        cls_total = jnp.sum(poscls_s[...]) - jnp.sum(sel)
        num_pos = jnp.sum(k_raw).astype(jnp.float32) / 3.0
        loss = (jnp.sum(locrow_s[...]) + cls_total) / num_pos
        out_ref[...] = loss[None, None]


def kernel(loc_preds, loc_targets, cls_preds, cls_targets):
    tt = cls_targets.astype(jnp.int32).reshape(N, 1, A)
    out = pl.pallas_call(
        _body,
        grid=(N,),
        in_specs=[
            pl.BlockSpec((1, A, 4), lambda n: (n, 0, 0)),
            pl.BlockSpec((1, A, 4), lambda n: (n, 0, 0)),
            pl.BlockSpec((1, C, A), lambda n: (n, 0, 0)),
            pl.BlockSpec((1, 1, A), lambda n: (n, 0, 0)),
        ],
        out_specs=pl.BlockSpec((1, 1), lambda n: (0, 0)),
        out_shape=jax.ShapeDtypeStruct((1, 1), jnp.float32),
        scratch_shapes=[
            pltpu.VMEM((N, A), jnp.uint32),
            pltpu.VMEM((N, A), jnp.float32),
            pltpu.VMEM((N, 1), jnp.int32),
            pltpu.VMEM((N, 1), jnp.float32),
            pltpu.VMEM((N, 1), jnp.float32),
        ],
    )(loc_preds, loc_targets, cls_preds, tt)
    return out[0, 0]
